# baseline (speedup 1.0000x reference)
import functools

import jax
import jax.numpy as jnp
from jax import lax
from jax.experimental import pallas as pl
from jax.experimental.pallas import tpu as pltpu
from jax.experimental.pallas import tpu_sc as plsc

VOCAB = 30522
HIDDEN = 768
MAX_POS = 2048
SF_LEVEL = 8
N_ETYPE = 16
B, S = 4, 2048
EPS = 1e-12

NTOK = B * S


_NC = 2
_NS = 16
_NW = _NC * _NS
_CHUNK = 64


@functools.cache
def _make_sc_gather(ntok):
    rows_per_w = ntok // _NW
    nch = rows_per_w // _CHUNK
    mesh = plsc.VectorSubcoreMesh(core_axis_name="c", subcore_axis_name="s")

    @functools.partial(
        pl.kernel,
        mesh=mesh,
        out_type=jax.ShapeDtypeStruct((ntok, HIDDEN), jnp.float32),
        scratch_types=[
            pltpu.VMEM((rows_per_w,), jnp.int32),
            pltpu.VMEM((2, _CHUNK, HIDDEN), jnp.float32),
            pltpu.SemaphoreType.DMA,
            pltpu.SemaphoreType.DMA,
            pltpu.SemaphoreType.DMA,
            pltpu.SemaphoreType.DMA,
        ],
    )
    def _sc_gather(ids_hbm, table_hbm, out_hbm, idx_v, rows_v, g0, g1, w0, w1):
        wid = lax.axis_index("s") * _NC + lax.axis_index("c")
        base = wid * rows_per_w
        pltpu.sync_copy(ids_hbm.at[pl.ds(base, rows_per_w)], idx_v)

        gsems = (g0, g1)
        wsems = (w0, w1)
        gathers = [None] * nch
        writes = [None] * nch

        def _issue_gather(ci):
            return pltpu.async_copy(
                table_hbm.at[idx_v.at[pl.ds(ci * _CHUNK, _CHUNK)]],
                rows_v.at[ci % 2],
                gsems[ci % 2],
            )

        gathers[0] = _issue_gather(0)
        for ci in range(nch):
            if ci + 1 < nch:
                if ci - 1 >= 0:
                    writes[ci - 1].wait()
                gathers[ci + 1] = _issue_gather(ci + 1)
            gathers[ci].wait()
            writes[ci] = pltpu.async_copy(
                rows_v.at[ci % 2],
                out_hbm.at[pl.ds(base + ci * _CHUNK, _CHUNK)],
                wsems[ci % 2],
            )
        writes[nch - 2].wait()
        writes[nch - 1].wait()

    return _sc_gather


_T = 1024
_NBLK = NTOK // _T
_SBLK = S // _T
_NSMALL = 64

_OFF_TT = 0
_OFF_ME = 2
_OFF_MT = 4
_OFF_SFE = 6
_OFF_SFT = 14
_OFF_ET = 22


def _tc_core(g_ref, p_ref, idx_ref, small_ref, gb_ref, out_ref):
    x = g_ref[...] + p_ref[...]

    row = lax.broadcasted_iota(jnp.int32, (_NSMALL, _T), 0)

    def onehot_t(i, off):
        idx = idx_ref[0, i:i + 1, :]
        return row == idx + off

    tt = (idx_ref[0, 0:1, :] > 0).astype(jnp.int32)
    oh = (row == tt + _OFF_TT)
    oh |= onehot_t(1, _OFF_ME)
    oh |= onehot_t(2, _OFF_MT)
    oh |= onehot_t(3, _OFF_SFE)
    oh |= onehot_t(4, _OFF_SFT)
    oh |= onehot_t(5, _OFF_ET)
    ohb = oh.astype(jnp.bfloat16)

    small = small_ref[...]
    hi = small.astype(jnp.bfloat16)
    lo = (small - hi.astype(jnp.float32)).astype(jnp.bfloat16)
    dn = (((0,), (0,)), ((), ()))
    aux = lax.dot_general(ohb, hi, dn, preferred_element_type=jnp.float32)
    aux += lax.dot_general(ohb, lo, dn, preferred_element_type=jnp.float32)
    x = x + aux

    mu = jnp.mean(x, axis=-1, keepdims=True)
    xc = x - mu
    var = jnp.mean(xc * xc, axis=-1, keepdims=True)
    y = xc * lax.rsqrt(var + EPS)
    out_ref[...] = y * gb_ref[0:1, :] + gb_ref[1:2, :]


def _tc_call(gathered, pos_emb, idx8, small, gb):
    tok = lambda sb, b: b * _SBLK + sb
    return pl.pallas_call(
        _tc_core,
        grid=(_SBLK, _NBLK // _SBLK),
        in_specs=[
            pl.BlockSpec((_T, HIDDEN), lambda sb, b: (tok(sb, b), 0)),
            pl.BlockSpec((_T, HIDDEN), lambda sb, b: (sb, 0)),
            pl.BlockSpec((1, 8, _T), lambda sb, b: (tok(sb, b), 0, 0)),
            pl.BlockSpec((_NSMALL, HIDDEN), lambda sb, b: (0, 0)),
            pl.BlockSpec((8, HIDDEN), lambda sb, b: (0, 0)),
        ],
        out_specs=pl.BlockSpec((_T, HIDDEN), lambda sb, b: (tok(sb, b), 0)),
        out_shape=jax.ShapeDtypeStruct((NTOK, HIDDEN), jnp.float32),
    )(gathered, pos_emb, idx8, small, gb)


def kernel(input_ids, token_type_ids, match_entity, sf_entity, match_token,
           sf_token, etype_ids, word_emb, token_type_emb, pos_emb,
           match_entity_emb, sf_entity_emb, match_token_emb, sf_token_emb,
           etype_emb, gamma, beta):
    ids = input_ids.reshape(NTOK).astype(jnp.int32)
    gathered = _make_sc_gather(NTOK)(ids, word_emb)

    flat = lambda a: a.reshape(NTOK).astype(jnp.int32)
    idx8 = jnp.stack(
        [flat(token_type_ids), flat(match_entity), flat(match_token),
         flat(sf_entity), flat(sf_token), flat(etype_ids),
         flat(etype_ids), flat(etype_ids)], axis=0)
    idx8 = idx8.reshape(8, _NBLK, _T).transpose(1, 0, 2)

    small = jnp.zeros((_NSMALL, HIDDEN), jnp.float32)
    small = small.at[_OFF_TT:_OFF_TT + 2].set(token_type_emb)
    small = small.at[_OFF_ME:_OFF_ME + 2].set(match_entity_emb)
    small = small.at[_OFF_MT:_OFF_MT + 2].set(match_token_emb)
    small = small.at[_OFF_SFE:_OFF_SFE + SF_LEVEL].set(sf_entity_emb)
    small = small.at[_OFF_SFT:_OFF_SFT + SF_LEVEL].set(sf_token_emb)
    small = small.at[_OFF_ET:_OFF_ET + N_ETYPE].set(etype_emb)

    gb = jnp.zeros((8, HIDDEN), jnp.float32)
    gb = gb.at[0].set(gamma).at[1].set(beta)

    out = _tc_call(gathered, pos_emb, idx8, small, gb)
    return out.reshape(B, S, HIDDEN)


# speedup vs baseline: 1.0113x; 1.0113x over previous
import functools

import jax
import jax.numpy as jnp
from jax import lax
from jax.experimental import pallas as pl
from jax.experimental.pallas import tpu as pltpu
from jax.experimental.pallas import tpu_sc as plsc

VOCAB = 30522
HIDDEN = 768
MAX_POS = 2048
SF_LEVEL = 8
N_ETYPE = 16
B, S = 4, 2048
EPS = 1e-12

NTOK = B * S


_NC = 2
_NS = 16
_NW = _NC * _NS
_CHUNK = 64


@functools.cache
def _make_sc_gather(ntok):
    rows_per_w = ntok // _NW
    nch = rows_per_w // _CHUNK
    mesh = plsc.VectorSubcoreMesh(core_axis_name="c", subcore_axis_name="s")

    @functools.partial(
        pl.kernel,
        mesh=mesh,
        out_type=jax.ShapeDtypeStruct((ntok, HIDDEN), jnp.float32),
        scratch_types=[
            pltpu.VMEM((rows_per_w,), jnp.int32),
            pltpu.VMEM((2, _CHUNK, HIDDEN), jnp.float32),
            pltpu.SemaphoreType.DMA,
            pltpu.SemaphoreType.DMA,
            pltpu.SemaphoreType.DMA,
            pltpu.SemaphoreType.DMA,
        ],
    )
    def _sc_gather(ids_hbm, table_hbm, out_hbm, idx_v, rows_v, g0, g1, w0, w1):
        wid = lax.axis_index("s") * _NC + lax.axis_index("c")
        base = wid * rows_per_w
        pltpu.sync_copy(ids_hbm.at[pl.ds(base, rows_per_w)], idx_v)

        gsems = (g0, g1)
        wsems = (w0, w1)
        gathers = [None] * nch
        writes = [None] * nch

        def _issue_gather(ci):
            return pltpu.async_copy(
                table_hbm.at[idx_v.at[pl.ds(ci * _CHUNK, _CHUNK)]],
                rows_v.at[ci % 2],
                gsems[ci % 2],
            )

        gathers[0] = _issue_gather(0)
        for ci in range(nch):
            if ci + 1 < nch:
                if ci - 1 >= 0:
                    writes[ci - 1].wait()
                gathers[ci + 1] = _issue_gather(ci + 1)
            gathers[ci].wait()
            writes[ci] = pltpu.async_copy(
                rows_v.at[ci % 2],
                out_hbm.at[pl.ds(base + ci * _CHUNK, _CHUNK)],
                wsems[ci % 2],
            )
        writes[nch - 2].wait()
        writes[nch - 1].wait()

    return _sc_gather


_T = 1024
_NBLK = NTOK // _T
_SBLK = S // _T
_NSMALL = 40

_OFF_TT = 0
_OFF_ME = 2
_OFF_MT = 4
_OFF_SFE = 6
_OFF_SFT = 14
_OFF_ET = 22


def _tc_core(g_ref, p_ref, idx_ref, small_ref, gb_ref, out_ref):
    x = g_ref[...] + p_ref[...]

    row = lax.broadcasted_iota(jnp.int32, (_NSMALL, _T), 0)

    def onehot_t(i, off):
        idx = idx_ref[0, i:i + 1, :]
        return row == idx + off

    tt = (idx_ref[0, 0:1, :] > 0).astype(jnp.int32)
    oh = (row == tt + _OFF_TT)
    oh |= onehot_t(1, _OFF_ME)
    oh |= onehot_t(2, _OFF_MT)
    oh |= onehot_t(3, _OFF_SFE)
    oh |= onehot_t(4, _OFF_SFT)
    oh |= onehot_t(5, _OFF_ET)
    ohb = oh.astype(jnp.bfloat16)

    small = small_ref[...]
    hi = small.astype(jnp.bfloat16)
    lo = (small - hi.astype(jnp.float32)).astype(jnp.bfloat16)
    dn = (((0,), (0,)), ((), ()))
    aux = lax.dot_general(ohb, hi, dn, preferred_element_type=jnp.float32)
    aux += lax.dot_general(ohb, lo, dn, preferred_element_type=jnp.float32)
    x = x + aux

    mu = jnp.mean(x, axis=-1, keepdims=True)
    xc = x - mu
    var = jnp.mean(xc * xc, axis=-1, keepdims=True)
    y = xc * lax.rsqrt(var + EPS)
    out_ref[...] = y * gb_ref[0:1, :] + gb_ref[1:2, :]


def _tc_call(gathered, pos_emb, idx8, small, gb):
    tok = lambda sb, b: b * _SBLK + sb
    return pl.pallas_call(
        _tc_core,
        grid=(_SBLK, _NBLK // _SBLK),
        in_specs=[
            pl.BlockSpec((_T, HIDDEN), lambda sb, b: (tok(sb, b), 0)),
            pl.BlockSpec((_T, HIDDEN), lambda sb, b: (sb, 0)),
            pl.BlockSpec((1, 8, _T), lambda sb, b: (tok(sb, b), 0, 0)),
            pl.BlockSpec((_NSMALL, HIDDEN), lambda sb, b: (0, 0)),
            pl.BlockSpec((8, HIDDEN), lambda sb, b: (0, 0)),
        ],
        out_specs=pl.BlockSpec((_T, HIDDEN), lambda sb, b: (tok(sb, b), 0)),
        out_shape=jax.ShapeDtypeStruct((NTOK, HIDDEN), jnp.float32),
    )(gathered, pos_emb, idx8, small, gb)


def kernel(input_ids, token_type_ids, match_entity, sf_entity, match_token,
           sf_token, etype_ids, word_emb, token_type_emb, pos_emb,
           match_entity_emb, sf_entity_emb, match_token_emb, sf_token_emb,
           etype_emb, gamma, beta):
    ids = input_ids.reshape(NTOK).astype(jnp.int32)
    gathered = _make_sc_gather(NTOK)(ids, word_emb)

    flat = lambda a: a.reshape(NTOK).astype(jnp.int32)
    idx8 = jnp.stack(
        [flat(token_type_ids), flat(match_entity), flat(match_token),
         flat(sf_entity), flat(sf_token), flat(etype_ids),
         flat(etype_ids), flat(etype_ids)], axis=0)
    idx8 = idx8.reshape(8, _NBLK, _T).transpose(1, 0, 2)

    small = jnp.zeros((_NSMALL, HIDDEN), jnp.float32)
    small = small.at[_OFF_TT:_OFF_TT + 2].set(token_type_emb)
    small = small.at[_OFF_ME:_OFF_ME + 2].set(match_entity_emb)
    small = small.at[_OFF_MT:_OFF_MT + 2].set(match_token_emb)
    small = small.at[_OFF_SFE:_OFF_SFE + SF_LEVEL].set(sf_entity_emb)
    small = small.at[_OFF_SFT:_OFF_SFT + SF_LEVEL].set(sf_token_emb)
    small = small.at[_OFF_ET:_OFF_ET + N_ETYPE].set(etype_emb)

    gb = jnp.zeros((8, HIDDEN), jnp.float32)
    gb = gb.at[0].set(gamma).at[1].set(beta)

    out = _tc_call(gathered, pos_emb, idx8, small, gb)
    return out.reshape(B, S, HIDDEN)
